# Initial kernel scaffold; baseline (speedup 1.0000x reference)
import jax
import jax.numpy as jnp
from jax.experimental import pallas as pl


def kernel(x, token_embedding):
    raise NotImplementedError("write your pallas kernel here")


# speedup vs baseline: 3.0835x; 3.0835x over previous
import functools

import jax
import jax.numpy as jnp
from jax import lax
from jax.experimental import pallas as pl
from jax.experimental.pallas import tpu as pltpu
from jax.experimental.pallas import tpu_sc as plsc

EMBED_DIM = 64
NC, NS = 2, 16
NW = NC * NS
STREAM_W = 128
CHUNK = 1024
NSTREAM = CHUNK // STREAM_W


def _make_lookup(total_rows: int):
    per_w = total_rows // NW
    nchunk = per_w // CHUNK
    mesh = plsc.VectorSubcoreMesh(core_axis_name="c", subcore_axis_name="s")

    @functools.partial(
        pl.kernel,
        mesh=mesh,
        out_type=jax.ShapeDtypeStruct((total_rows, EMBED_DIM), jnp.float32),
        compiler_params=pltpu.CompilerParams(use_tc_tiling_on_sc=False),
        scratch_types=[
            pltpu.VMEM((NSTREAM, STREAM_W), jnp.int32),
            pltpu.VMEM((CHUNK, EMBED_DIM), jnp.float32),
            pltpu.SemaphoreType.DMA,
        ],
    )
    def lookup(x_hbm, tab_hbm, out_hbm, idx_v, rows_v, gsem):
        wid = lax.axis_index("s") * NC + lax.axis_index("c")
        row_base = wid * per_w

        def body(c, _):
            off = pl.multiple_of(row_base + c * CHUNK, CHUNK)
            pltpu.sync_copy(
                x_hbm.at[pl.ds(pl.multiple_of(off // STREAM_W, NSTREAM), NSTREAM)],
                idx_v,
            )
            for j in range(NSTREAM):
                pltpu.async_copy(
                    tab_hbm.at[idx_v.at[j]],
                    rows_v.at[pl.ds(j * STREAM_W, STREAM_W)],
                    gsem,
                )
            for j in range(NSTREAM):
                pltpu.make_async_copy(
                    tab_hbm.at[idx_v.at[j]],
                    rows_v.at[pl.ds(j * STREAM_W, STREAM_W)],
                    gsem,
                ).wait()
            pltpu.sync_copy(rows_v, out_hbm.at[pl.ds(off, CHUNK)])
            return ()

        lax.fori_loop(0, nchunk, body, (), unroll=False)

    return lookup


def kernel(x, token_embedding):
    B, L = x.shape
    total = B * L
    idx = x.astype(jnp.int32).reshape(total // STREAM_W, STREAM_W)
    out = _make_lookup(total)(idx, token_embedding)
    return out.reshape(B, L, EMBED_DIM)
